# baseline (speedup 1.0000x reference)
import functools

import jax
import jax.numpy as jnp
from jax import lax
from jax.experimental import pallas as pl
from jax.experimental.pallas import tpu as pltpu
from jax.experimental.pallas import tpu_sc as plsc

_LANES = 16

_I32_SIGN_INT = -2147483648
_I32_MAG_INT = 0x7FFFFFFF


@functools.lru_cache(maxsize=None)
def _make_sc_lookup(vocab_n: int, batch_n: int):
    info = plsc.get_sparse_core_info()
    num_cores, num_subcores = info.num_cores, info.num_subcores
    num_workers = num_cores * num_subcores
    chunk = batch_n // num_workers
    assert chunk % (8 * _LANES) == 0 and chunk * num_workers == batch_n
    assert vocab_n % _LANES == 0
    samp_n = vocab_n // _LANES
    steps1 = max(1, (samp_n - 1).bit_length())
    nblk = chunk // 128
    mesh = plsc.VectorSubcoreMesh(core_axis_name="c", subcore_axis_name="s")

    @functools.partial(
        pl.kernel,
        out_type=jax.ShapeDtypeStruct((batch_n,), jnp.int32),
        mesh=mesh,
        scratch_types=[
            pltpu.VMEM((samp_n,), jnp.int32),
            pltpu.VMEM((chunk,), jnp.int32),
            pltpu.VMEM((chunk,), jnp.int32),
            pltpu.VMEM((chunk, _LANES), jnp.int32),
            pltpu.VMEM((chunk,), jnp.int32),
            pltpu.SemaphoreType.DMA,
            pltpu.SemaphoreType.DMA,
        ],
        compiler_params=pltpu.CompilerParams(needs_layout_passes=False,
                                             use_tc_tiling_on_sc=False),
    )
    def lookup(samp_hbm, vocab2d_hbm, ids_hbm, out_hbm,
               samp_v, ids_v, rows_v, win_v, res_v, sem, sem2):
        wid = lax.axis_index("s") * num_cores + lax.axis_index("c")
        base = wid * chunk
        c_samp = pltpu.async_copy(samp_hbm, samp_v, sem2)
        c_ids = pltpu.async_copy(ids_hbm.at[pl.ds(base, chunk)], ids_v, sem2)
        c_samp.wait()
        c_ids.wait()

        @pl.loop(0, nblk // 2)
        def _phase1(i):
            off0 = pl.multiple_of(i * 256, 256)
            ids = [ids_v[pl.ds(off0 + k * _LANES, _LANES)] for k in range(16)]
            lo = [jnp.full((_LANES,), -1, jnp.int32) for _ in range(16)]
            for s in range(steps1):
                step = 1 << (steps1 - 1 - s)
                t = [jnp.minimum(l + step, samp_n - 1) for l in lo]
                v = [plsc.load_gather(samp_v, [tk]) for tk in t]
                lo = [jnp.where(vk <= idk, tk, l)
                      for vk, idk, tk, l in zip(v, ids, t, lo)]
            for k, l in enumerate(lo):
                rows_v[pl.ds(off0 + k * _LANES, _LANES)] = jnp.maximum(l, 0)
            for h in range(2):
                b = pl.multiple_of(off0 + h * 128, 128)
                pltpu.async_copy(vocab2d_hbm.at[rows_v.at[pl.ds(b, 128)]],
                                 win_v.at[pl.ds(b, 128)], sem)

        for i in range(nblk):
            pltpu.make_async_copy(
                vocab2d_hbm.at[rows_v.at[pl.ds(i * 128, 128)]],
                win_v.at[pl.ds(i * 128, 128)], sem).wait()

        @pl.loop(0, nblk // 2)
        def _phase3(b):
            for j in range(16):
                off = pl.multiple_of(b * 256 + j * _LANES, _LANES)
                ids = ids_v[pl.ds(off, _LANES)]
                r = rows_v[pl.ds(off, _LANES)]
                idrow = lax.iota(jnp.int32, _LANES) + off
                lo = jnp.full((_LANES,), -1, jnp.int32)
                for step in (16, 8, 4, 2, 1):
                    t = jnp.minimum(lo + step, _LANES - 1)
                    v = plsc.load_gather(win_v, [idrow, t])
                    lo = jnp.where(v < ids, t, lo)
                cnt = lo + 1
                pos = jnp.minimum(r * _LANES + cnt, vocab_n - 1)
                v_in = plsc.load_gather(
                    win_v, [idrow, jnp.minimum(cnt, _LANES - 1)])
                v_nxt = plsc.load_gather(
                    samp_v, [jnp.minimum(r + 1, samp_n - 1)])
                spill = (cnt == _LANES) & (r < samp_n - 1)
                vv = jnp.where(spill, v_nxt, v_in)
                res_v[pl.ds(off, _LANES)] = jnp.where(vv == ids, pos + 1, 0)

        pltpu.sync_copy(res_v, out_hbm.at[pl.ds(base, chunk)])

    return lookup


def _key_from_bits(b):
    return jnp.where(b < 0, b ^ jnp.int32(_I32_MAG_INT), b)


def _tc_stats_body(nbins, k1, k3, price_ref, bnd_ref, mv_ref,
                   clip_ref, disc_ref, norm_ref):
    p = price_ref[...]
    key = _key_from_bits(lax.bitcast_convert_type(p, jnp.int32))
    mn_key = jnp.min(key)

    a1 = jnp.int32(0)
    a3 = jnp.int32(0)
    for bit in range(31, -1, -1):
        mval = 1 << bit
        if mval >= 2**31:
            mval -= 2**32
        m = jnp.int32(mval)
        t1 = a1 | m
        t3 = a3 | m
        ts1 = t1 ^ jnp.int32(_I32_SIGN_INT)
        ts3 = t3 ^ jnp.int32(_I32_SIGN_INT)
        c = jnp.sum((key < ts1).astype(jnp.int32)
                    + ((key < ts3).astype(jnp.int32) << 16))
        c1 = c & jnp.int32(0xFFFF)
        c3 = c >> 16
        a1 = jnp.where(c1 <= k1, t1, a1)
        a3 = jnp.where(c3 <= k3, t3, a3)

    def key_to_f32(s):
        return lax.bitcast_convert_type(_key_from_bits(s), jnp.float32)

    q1 = key_to_f32(a1 ^ jnp.int32(_I32_SIGN_INT))
    q3 = key_to_f32(a3 ^ jnp.int32(_I32_SIGN_INT))
    mn = key_to_f32(mn_key)
    iqr = q3 - q1
    lower = jnp.maximum(q1 - 3.0 * iqr, mn)
    upper = q3 + 3.0 * iqr
    cp = jnp.clip(p, lower, upper)
    clip_ref[...] = cp
    norm_ref[...] = (cp - mv_ref[0]) / jnp.sqrt(mv_ref[1])

    acc = jnp.zeros(p.shape, jnp.int32)
    for j in range(nbins - 1):
        acc += (bnd_ref[j] <= cp).astype(jnp.int32)
    disc_ref[...] = acc


@functools.lru_cache(maxsize=None)
def _make_tc_stats(rows: int, cols: int, nbins: int):
    n = rows * cols
    k1 = (25 * (n - 1)) // 100
    k3 = (75 * (n - 1)) // 100
    return pl.pallas_call(
        functools.partial(_tc_stats_body, nbins, k1, k3),
        out_shape=(
            jax.ShapeDtypeStruct((rows, cols), jnp.float32),
            jax.ShapeDtypeStruct((rows, cols), jnp.int32),
            jax.ShapeDtypeStruct((rows, cols), jnp.float32),
        ),
        in_specs=[
            pl.BlockSpec(memory_space=pltpu.VMEM),
            pl.BlockSpec(memory_space=pltpu.SMEM),
            pl.BlockSpec(memory_space=pltpu.SMEM),
        ],
    )


def kernel(item_id, price, vocab, norm_mean, norm_var, bin_boundaries):
    batch_n = price.shape[0]
    vocab_n = vocab.shape[0]
    nbins = bin_boundaries.shape[0] + 1

    vocab2d = vocab.reshape(vocab_n // _LANES, _LANES)
    samp = vocab[::_LANES]
    int_item_id = _make_sc_lookup(vocab_n, batch_n)(samp, vocab2d, item_id)

    rows = batch_n // 128
    p2 = price.reshape(rows, 128)
    mv = jnp.stack([jnp.asarray(norm_mean, jnp.float32),
                    jnp.asarray(norm_var, jnp.float32)])
    clip2, disc2, norm2 = _make_tc_stats(rows, 128, nbins)(
        p2, bin_boundaries, mv)

    return (int_item_id,
            disc2.reshape(batch_n),
            norm2.reshape(batch_n),
            clip2.reshape(batch_n))


# speedup vs baseline: 1.0359x; 1.0359x over previous
import functools

import jax
import jax.numpy as jnp
from jax import lax
from jax.experimental import pallas as pl
from jax.experimental.pallas import tpu as pltpu
from jax.experimental.pallas import tpu_sc as plsc

_LANES = 16

_I32_SIGN_INT = -2147483648
_I32_MAG_INT = 0x7FFFFFFF


@functools.lru_cache(maxsize=None)
def _make_sc_lookup(vocab_n: int, batch_n: int):
    info = plsc.get_sparse_core_info()
    num_cores, num_subcores = info.num_cores, info.num_subcores
    num_workers = num_cores * num_subcores
    chunk = batch_n // num_workers
    assert chunk % (8 * _LANES) == 0 and chunk * num_workers == batch_n
    assert vocab_n % _LANES == 0
    samp_n = vocab_n // _LANES
    steps1 = max(1, (samp_n - 1).bit_length())
    nblk = chunk // 128
    mesh = plsc.VectorSubcoreMesh(core_axis_name="c", subcore_axis_name="s")

    @functools.partial(
        pl.kernel,
        out_type=jax.ShapeDtypeStruct((batch_n,), jnp.int32),
        mesh=mesh,
        scratch_types=[
            pltpu.VMEM((samp_n,), jnp.int32),
            pltpu.VMEM((chunk,), jnp.int32),
            pltpu.VMEM((nblk, 128), jnp.int32),
            pltpu.VMEM((chunk, _LANES), jnp.int32),
            pltpu.VMEM((chunk,), jnp.int32),
            pltpu.SemaphoreType.DMA,
            pltpu.SemaphoreType.DMA,
        ],
        compiler_params=pltpu.CompilerParams(needs_layout_passes=False,
                                             use_tc_tiling_on_sc=False),
    )
    def lookup(samp_hbm, vocab2d_hbm, ids_hbm, out_hbm,
               samp_v, ids_v, rows_v, win_v, res_v, sem, sem2):
        wid = lax.axis_index("s") * num_cores + lax.axis_index("c")
        base = wid * chunk
        c_samp = pltpu.async_copy(samp_hbm, samp_v, sem2)
        c_ids = pltpu.async_copy(ids_hbm.at[pl.ds(base, chunk)], ids_v, sem2)
        c_samp.wait()
        c_ids.wait()

        @pl.loop(0, nblk)
        def _phase1(i):
            off0 = pl.multiple_of(i * 128, 128)
            ids = [ids_v[pl.ds(off0 + k * _LANES, _LANES)] for k in range(8)]
            lo = [jnp.full((_LANES,), -1, jnp.int32) for _ in range(8)]
            for s in range(steps1):
                step = 1 << (steps1 - 1 - s)
                t = [jnp.minimum(l + step, samp_n - 1) for l in lo]
                v = [plsc.load_gather(samp_v, [tk]) for tk in t]
                lo = [jnp.where(vk <= idk, tk, l)
                      for vk, idk, tk, l in zip(v, ids, t, lo)]
            for k, l in enumerate(lo):
                rows_v[i, pl.ds(k * _LANES, _LANES)] = jnp.maximum(l, 0)
            pltpu.async_copy(vocab2d_hbm.at[rows_v.at[i]],
                             win_v.at[pl.ds(i * 128, 128)], sem)

        for i in range(nblk):
            pltpu.make_async_copy(vocab2d_hbm.at[rows_v.at[i]],
                                  win_v.at[pl.ds(i * 128, 128)], sem).wait()

        @pl.loop(0, nblk)
        def _phase3(b):
            for j in range(8):
                off = pl.multiple_of(b * 128 + j * _LANES, _LANES)
                ids = ids_v[pl.ds(off, _LANES)]
                r = rows_v[b, pl.ds(j * _LANES, _LANES)]
                idrow = lax.iota(jnp.int32, _LANES) + off
                lo = jnp.full((_LANES,), -1, jnp.int32)
                for step in (16, 8, 4, 2, 1):
                    t = jnp.minimum(lo + step, _LANES - 1)
                    v = plsc.load_gather(win_v, [idrow, t])
                    lo = jnp.where(v < ids, t, lo)
                cnt = lo + 1
                pos = jnp.minimum(r * _LANES + cnt, vocab_n - 1)
                v_in = plsc.load_gather(
                    win_v, [idrow, jnp.minimum(cnt, _LANES - 1)])
                v_nxt = plsc.load_gather(
                    samp_v, [jnp.minimum(r + 1, samp_n - 1)])
                spill = (cnt == _LANES) & (r < samp_n - 1)
                vv = jnp.where(spill, v_nxt, v_in)
                res_v[pl.ds(off, _LANES)] = jnp.where(vv == ids, pos + 1, 0)

        pltpu.sync_copy(res_v, out_hbm.at[pl.ds(base, chunk)])

    return lookup


def _key_from_bits(b):
    return jnp.where(b < 0, b ^ jnp.int32(_I32_MAG_INT), b)


def _tc_stats_body(nbins, k1, k3, price_ref, bnd_ref, mv_ref,
                   clip_ref, disc_ref, norm_ref):
    p = price_ref[...]
    key = _key_from_bits(lax.bitcast_convert_type(p, jnp.int32))
    mn_key = jnp.min(key)

    a1 = jnp.int32(0)
    a3 = jnp.int32(0)
    for bit in range(31, -1, -1):
        mval = 1 << bit
        if mval >= 2**31:
            mval -= 2**32
        m = jnp.int32(mval)
        t1 = a1 | m
        t3 = a3 | m
        ts1 = t1 ^ jnp.int32(_I32_SIGN_INT)
        ts3 = t3 ^ jnp.int32(_I32_SIGN_INT)
        c = jnp.sum((key < ts1).astype(jnp.int32)
                    + ((key < ts3).astype(jnp.int32) << 16))
        c1 = c & jnp.int32(0xFFFF)
        c3 = c >> 16
        a1 = jnp.where(c1 <= k1, t1, a1)
        a3 = jnp.where(c3 <= k3, t3, a3)

    def key_to_f32(s):
        return lax.bitcast_convert_type(_key_from_bits(s), jnp.float32)

    q1 = key_to_f32(a1 ^ jnp.int32(_I32_SIGN_INT))
    q3 = key_to_f32(a3 ^ jnp.int32(_I32_SIGN_INT))
    mn = key_to_f32(mn_key)
    iqr = q3 - q1
    lower = jnp.maximum(q1 - 3.0 * iqr, mn)
    upper = q3 + 3.0 * iqr
    cp = jnp.clip(p, lower, upper)
    clip_ref[...] = cp
    norm_ref[...] = (cp - mv_ref[0]) / jnp.sqrt(mv_ref[1])

    acc = jnp.zeros(p.shape, jnp.int32)
    for j in range(nbins - 1):
        acc += (bnd_ref[j] <= cp).astype(jnp.int32)
    disc_ref[...] = acc


@functools.lru_cache(maxsize=None)
def _make_tc_stats(rows: int, cols: int, nbins: int):
    n = rows * cols
    k1 = (25 * (n - 1)) // 100
    k3 = (75 * (n - 1)) // 100
    return pl.pallas_call(
        functools.partial(_tc_stats_body, nbins, k1, k3),
        out_shape=(
            jax.ShapeDtypeStruct((rows, cols), jnp.float32),
            jax.ShapeDtypeStruct((rows, cols), jnp.int32),
            jax.ShapeDtypeStruct((rows, cols), jnp.float32),
        ),
        in_specs=[
            pl.BlockSpec(memory_space=pltpu.VMEM),
            pl.BlockSpec(memory_space=pltpu.SMEM),
            pl.BlockSpec(memory_space=pltpu.SMEM),
        ],
    )


def kernel(item_id, price, vocab, norm_mean, norm_var, bin_boundaries):
    batch_n = price.shape[0]
    vocab_n = vocab.shape[0]
    nbins = bin_boundaries.shape[0] + 1

    vocab2d = vocab.reshape(vocab_n // _LANES, _LANES)
    samp = vocab[::_LANES]
    int_item_id = _make_sc_lookup(vocab_n, batch_n)(samp, vocab2d, item_id)

    rows = batch_n // 128
    p2 = price.reshape(rows, 128)
    mv = jnp.stack([jnp.asarray(norm_mean, jnp.float32),
                    jnp.asarray(norm_var, jnp.float32)])
    clip2, disc2, norm2 = _make_tc_stats(rows, 128, nbins)(
        p2, bin_boundaries, mv)

    return (int_item_id,
            disc2.reshape(batch_n),
            norm2.reshape(batch_n),
            clip2.reshape(batch_n))


# speedup vs baseline: 1.1068x; 1.0685x over previous
import functools

import jax
import jax.numpy as jnp
from jax import lax
from jax.experimental import pallas as pl
from jax.experimental.pallas import tpu as pltpu
from jax.experimental.pallas import tpu_sc as plsc

_LANES = 16

_I32_SIGN_INT = -2147483648
_I32_MAG_INT = 0x7FFFFFFF


@functools.lru_cache(maxsize=None)
def _make_sc_lookup(vocab_n: int, batch_n: int):
    info = plsc.get_sparse_core_info()
    num_cores, num_subcores = info.num_cores, info.num_subcores
    num_workers = num_cores * num_subcores
    chunk = batch_n // num_workers
    assert chunk % (8 * _LANES) == 0 and chunk * num_workers == batch_n
    assert vocab_n % _LANES == 0
    samp_n = vocab_n // _LANES
    steps1 = max(1, (samp_n - 1).bit_length())
    nblk = chunk // 128
    mesh = plsc.VectorSubcoreMesh(core_axis_name="c", subcore_axis_name="s")

    @functools.partial(
        pl.kernel,
        out_type=jax.ShapeDtypeStruct((batch_n,), jnp.int32),
        mesh=mesh,
        scratch_types=[
            pltpu.VMEM((samp_n,), jnp.int32),
            pltpu.VMEM((chunk,), jnp.int32),
            pltpu.VMEM((nblk, 128), jnp.int32),
            pltpu.VMEM((chunk, _LANES), jnp.int32),
            pltpu.VMEM((chunk,), jnp.int32),
            pltpu.SemaphoreType.DMA,
            pltpu.SemaphoreType.DMA,
        ],
        compiler_params=pltpu.CompilerParams(needs_layout_passes=False,
                                             use_tc_tiling_on_sc=False),
    )
    def lookup(samp_hbm, vocab2d_hbm, ids_hbm, out_hbm,
               samp_v, ids_v, rows_v, win_v, res_v, sem, sem2):
        wid = lax.axis_index("s") * num_cores + lax.axis_index("c")
        base = wid * chunk
        c_samp = pltpu.async_copy(samp_hbm, samp_v, sem2)
        c_ids = pltpu.async_copy(ids_hbm.at[pl.ds(base, chunk)], ids_v, sem2)
        c_samp.wait()
        c_ids.wait()

        @pl.loop(0, nblk)
        def _phase1(i):
            off0 = pl.multiple_of(i * 128, 128)
            ids = [ids_v[pl.ds(off0 + k * _LANES, _LANES)] for k in range(8)]
            lo = [jnp.full((_LANES,), -1, jnp.int32) for _ in range(8)]
            for s in range(steps1):
                step = 1 << (steps1 - 1 - s)
                t = [jnp.minimum(l + step, samp_n - 1) for l in lo]
                v = [plsc.load_gather(samp_v, [tk]) for tk in t]
                lo = [jnp.where(vk <= idk, tk, l)
                      for vk, idk, tk, l in zip(v, ids, t, lo)]
            for k, l in enumerate(lo):
                rows_v[i, pl.ds(k * _LANES, _LANES)] = jnp.maximum(l, 0)
            pltpu.async_copy(vocab2d_hbm.at[rows_v.at[i]],
                             win_v.at[pl.ds(i * 128, 128)], sem)

        for i in range(nblk):
            pltpu.make_async_copy(vocab2d_hbm.at[rows_v.at[i]],
                                  win_v.at[pl.ds(i * 128, 128)], sem).wait()

        @pl.loop(0, nblk)
        def _phase3(b):
            offs = [pl.multiple_of(b * 128 + j * _LANES, _LANES)
                    for j in range(8)]
            ids = [ids_v[pl.ds(off, _LANES)] for off in offs]
            r = [rows_v[b, pl.ds(j * _LANES, _LANES)] for j in range(8)]
            idrow = [lax.iota(jnp.int32, _LANES) + off for off in offs]
            lo = [jnp.full((_LANES,), -1, jnp.int32) for _ in range(8)]
            for step in (16, 8, 4, 2, 1):
                t = [jnp.minimum(l + step, _LANES - 1) for l in lo]
                v = [plsc.load_gather(win_v, [idrow[j], t[j]])
                     for j in range(8)]
                lo = [jnp.where(vj < idj, tj, l)
                      for vj, idj, tj, l in zip(v, ids, t, lo)]
            cnt = [l + 1 for l in lo]
            v_in = [plsc.load_gather(
                win_v, [idrow[j], jnp.minimum(cnt[j], _LANES - 1)])
                for j in range(8)]
            v_nxt = [plsc.load_gather(
                samp_v, [jnp.minimum(rj + 1, samp_n - 1)]) for rj in r]
            for j in range(8):
                pos = jnp.minimum(r[j] * _LANES + cnt[j], vocab_n - 1)
                spill = (cnt[j] == _LANES) & (r[j] < samp_n - 1)
                vv = jnp.where(spill, v_nxt[j], v_in[j])
                res_v[pl.ds(offs[j], _LANES)] = jnp.where(
                    vv == ids[j], pos + 1, 0)

        pltpu.sync_copy(res_v, out_hbm.at[pl.ds(base, chunk)])

    return lookup


def _key_from_bits(b):
    return jnp.where(b < 0, b ^ jnp.int32(_I32_MAG_INT), b)


def _tc_stats_body(nbins, k1, k3, price_ref, bnd_ref, mv_ref,
                   clip_ref, disc_ref, norm_ref):
    p = price_ref[...]
    key = _key_from_bits(lax.bitcast_convert_type(p, jnp.int32))
    mn_key = jnp.min(key)

    a1 = jnp.int32(0)
    a3 = jnp.int32(0)
    for bit in range(31, -1, -1):
        mval = 1 << bit
        if mval >= 2**31:
            mval -= 2**32
        m = jnp.int32(mval)
        t1 = a1 | m
        t3 = a3 | m
        ts1 = t1 ^ jnp.int32(_I32_SIGN_INT)
        ts3 = t3 ^ jnp.int32(_I32_SIGN_INT)
        c = jnp.sum((key < ts1).astype(jnp.int32)
                    + ((key < ts3).astype(jnp.int32) << 16))
        c1 = c & jnp.int32(0xFFFF)
        c3 = c >> 16
        a1 = jnp.where(c1 <= k1, t1, a1)
        a3 = jnp.where(c3 <= k3, t3, a3)

    def key_to_f32(s):
        return lax.bitcast_convert_type(_key_from_bits(s), jnp.float32)

    q1 = key_to_f32(a1 ^ jnp.int32(_I32_SIGN_INT))
    q3 = key_to_f32(a3 ^ jnp.int32(_I32_SIGN_INT))
    mn = key_to_f32(mn_key)
    iqr = q3 - q1
    lower = jnp.maximum(q1 - 3.0 * iqr, mn)
    upper = q3 + 3.0 * iqr
    cp = jnp.clip(p, lower, upper)
    clip_ref[...] = cp
    norm_ref[...] = (cp - mv_ref[0]) / jnp.sqrt(mv_ref[1])

    acc = jnp.zeros(p.shape, jnp.int32)
    for j in range(nbins - 1):
        acc += (bnd_ref[j] <= cp).astype(jnp.int32)
    disc_ref[...] = acc


@functools.lru_cache(maxsize=None)
def _make_tc_stats(rows: int, cols: int, nbins: int):
    n = rows * cols
    k1 = (25 * (n - 1)) // 100
    k3 = (75 * (n - 1)) // 100
    return pl.pallas_call(
        functools.partial(_tc_stats_body, nbins, k1, k3),
        out_shape=(
            jax.ShapeDtypeStruct((rows, cols), jnp.float32),
            jax.ShapeDtypeStruct((rows, cols), jnp.int32),
            jax.ShapeDtypeStruct((rows, cols), jnp.float32),
        ),
        in_specs=[
            pl.BlockSpec(memory_space=pltpu.VMEM),
            pl.BlockSpec(memory_space=pltpu.SMEM),
            pl.BlockSpec(memory_space=pltpu.SMEM),
        ],
    )


def kernel(item_id, price, vocab, norm_mean, norm_var, bin_boundaries):
    batch_n = price.shape[0]
    vocab_n = vocab.shape[0]
    nbins = bin_boundaries.shape[0] + 1

    vocab2d = vocab.reshape(vocab_n // _LANES, _LANES)
    samp = vocab[::_LANES]
    int_item_id = _make_sc_lookup(vocab_n, batch_n)(samp, vocab2d, item_id)

    rows = batch_n // 128
    p2 = price.reshape(rows, 128)
    mv = jnp.stack([jnp.asarray(norm_mean, jnp.float32),
                    jnp.asarray(norm_var, jnp.float32)])
    clip2, disc2, norm2 = _make_tc_stats(rows, 128, nbins)(
        p2, bin_boundaries, mv)

    return (int_item_id,
            disc2.reshape(batch_n),
            norm2.reshape(batch_n),
            clip2.reshape(batch_n))
